# baseline (speedup 1.0000x reference)
import functools

import jax
import jax.numpy as jnp
from jax import lax
from jax.experimental import pallas as pl

K = 1024
EMA_DECAY = 0.99
TOL = 1e-4
EPS = 1e-5
MAX_ITERS = 4

_BM = 1024


def _dist_argmin_body(x_ref, c_ref, o_ref):
    x = x_ref[...]
    c = c_ref[...]
    x2 = jnp.sum(x * x, axis=1, keepdims=True)
    c2 = jnp.sum(c * c, axis=1, keepdims=True).T
    xc = lax.dot_general(x, c, (((1,), (1,)), ((), ())),
                         preferred_element_type=jnp.float32)
    d2 = x2 + c2 - 2.0 * xc
    d = jnp.sqrt(jnp.maximum(d2, 0.0))
    mn = jnp.min(d, axis=1, keepdims=True)
    iota = lax.broadcasted_iota(jnp.int32, d.shape, 1)
    lab = jnp.min(jnp.where(d == mn, iota, jnp.int32(K)), axis=1)
    o_ref[0, 0, :] = lab


def _dist_argmin(xf, centers):
    n = xf.shape[0]
    nb = n // _BM
    out = pl.pallas_call(
        _dist_argmin_body,
        grid=(nb,),
        in_specs=[
            pl.BlockSpec((_BM, 256), lambda i: (i, 0)),
            pl.BlockSpec((K, 256), lambda i: (0, 0)),
        ],
        out_specs=pl.BlockSpec((1, 1, _BM), lambda i: (i, 0, 0)),
        out_shape=jax.ShapeDtypeStruct((nb, 1, _BM), jnp.int32),
    )(xf, centers)
    return out.reshape(-1)


def _update_body(ema_ref, ps_ref, pc_ref, oe_ref, oerr_ref):
    ema = ema_ref[...]
    sums = ps_ref[0] + ps_ref[1]
    counts = pc_ref[0, :, 0:1] + pc_ref[1, :, 0:1]
    centers = sums / (counts + EPS)
    centers = jnp.where(counts == 0.0, ema, centers)
    ema_new = ema * EMA_DECAY + (1.0 - EMA_DECAY) * centers
    diff = ema_new - centers
    oe_ref[...] = ema_new
    oerr_ref[...] = jnp.broadcast_to(jnp.sqrt(jnp.sum(diff * diff)), (1, 1))


def _update(ema, psums, pcnts):
    return pl.pallas_call(
        _update_body,
        out_shape=(
            jax.ShapeDtypeStruct((K, 256), jnp.float32),
            jax.ShapeDtypeStruct((1, 1), jnp.float32),
        ),
    )(ema, psums, pcnts)


def _segment_stats(xf, labels):
    n = xf.shape[0]
    half = n // 2
    s0 = jax.ops.segment_sum(xf[:half], labels[:half], num_segments=K)
    s1 = jax.ops.segment_sum(xf[half:], labels[half:], num_segments=K)
    c0 = jax.ops.segment_sum(jnp.ones((half,), jnp.float32), labels[:half],
                             num_segments=K)
    c1 = jax.ops.segment_sum(jnp.ones((half,), jnp.float32), labels[half:],
                             num_segments=K)
    psums = jnp.stack([s0, s1])
    pcnts = jnp.stack([c0, c1])[:, :, None] * jnp.ones((1, 1, 16), jnp.float32)
    return psums, pcnts


def kernel(x):
    b, c, t = x.shape
    xf = jnp.transpose(x, (0, 2, 1)).reshape(-1, c)
    n = b * t
    perm = jax.random.permutation(jax.random.key(1), n)
    ema0 = xf[perm[:K], :]

    def cond_fn(carry):
        i, ema, labels, broke = carry
        return jnp.logical_and(i < MAX_ITERS, jnp.logical_not(broke))

    def body_fn(carry):
        i, ema, labels, broke = carry
        labels = _dist_argmin(xf, ema)
        psums, pcnts = _segment_stats(xf, labels)
        ema_new, err = _update(ema, psums, pcnts)
        broke = err[0, 0] < TOL
        return (i + 1, ema_new, labels, broke)

    init = (jnp.asarray(0, jnp.int32), ema0,
            jnp.zeros((n,), jnp.int32), jnp.asarray(False))
    _, ema_f, labels, broke = lax.while_loop(cond_fn, body_fn, init)

    labels_else = _dist_argmin(xf, ema_f)
    return jnp.where(broke, labels, labels_else)


# speedup vs baseline: 1.1691x; 1.1691x over previous
import functools

import jax
import jax.numpy as jnp
from jax import lax
from jax.experimental import pallas as pl
from jax.experimental.pallas import tpu as pltpu
from jax.experimental.pallas import tpu_sc as plsc

K = 1024
EMA_DECAY = 0.99
TOL = 1e-4
EPS = 1e-5
MAX_ITERS = 4

N = 16384
C = 256
_BM = 1024
_NG = 16
_GW = C // _NG
_CHUNK = 1024
_NCHUNK = N // _CHUNK


def _dist_argmin_body(x_ref, c_ref, o_ref):
    x = x_ref[...]
    c = c_ref[...]
    x2 = jnp.sum(x * x, axis=1, keepdims=True)
    c2 = jnp.sum(c * c, axis=1, keepdims=True).T
    xc = lax.dot_general(x, c, (((1,), (1,)), ((), ())),
                         preferred_element_type=jnp.float32)
    d2 = x2 + c2 - 2.0 * xc
    d = jnp.sqrt(jnp.maximum(d2, 0.0))
    mn = jnp.min(d, axis=1, keepdims=True)
    iota = lax.broadcasted_iota(jnp.int32, d.shape, 1)
    lab = jnp.min(jnp.where(d == mn, iota, jnp.int32(K)), axis=1)
    o_ref[0, 0, :] = lab


def _dist_argmin(xf, centers):
    nb = N // _BM
    out = pl.pallas_call(
        _dist_argmin_body,
        grid=(nb,),
        in_specs=[
            pl.BlockSpec((_BM, C), lambda i: (i, 0)),
            pl.BlockSpec((K, C), lambda i: (0, 0)),
        ],
        out_specs=pl.BlockSpec((1, 1, _BM), lambda i: (i, 0, 0)),
        out_shape=jax.ShapeDtypeStruct((nb, 1, _BM), jnp.int32),
    )(xf, centers)
    return out.reshape(-1)


def _segsum_body(xg_hbm, lab_hbm, osum_hbm, ocnt_hbm, acc, cnt, xbuf, lbuf):
    cid = lax.axis_index("c")
    sid = lax.axis_index("s")
    zeros16 = jnp.zeros((16,), jnp.float32)
    ones16 = jnp.ones((16,), jnp.float32)

    pltpu.sync_copy(lab_hbm, lbuf)

    @pl.when(sid < 8)
    def _sums():
        g = cid * 8 + sid

        def zero_acc(i, carry):
            acc[pl.ds(i * 16, 16)] = zeros16
            return carry
        lax.fori_loop(0, K, zero_acc, 0)

        gbase = g * (N * _GW)

        def chunk_body(ch, carry):
            pltpu.sync_copy(
                xg_hbm.at[pl.ds(gbase + ch * (_CHUNK * _GW), _CHUNK * _GW)],
                xbuf)

            def grp(gr, c2):
                lv = lbuf[pl.ds(ch * _CHUNK + gr * 16, 16)]
                for j in range(16):
                    acc[pl.ds(lv[j] * 16, 16)] += (
                        xbuf[pl.ds((gr * 16 + j) * 16, 16)])
                return c2
            lax.fori_loop(0, _CHUNK // 16, grp, 0)
            return carry
        lax.fori_loop(0, _NCHUNK, chunk_body, 0)
        pltpu.sync_copy(acc, osum_hbm.at[g])

    @pl.when(sid >= 8)
    def _counts():
        w = cid * 8 + (sid - 8)

        def zero_cnt(i, carry):
            cnt[pl.ds(i * 16, 16)] = zeros16
            return carry
        lax.fori_loop(0, K, zero_cnt, 0)

        def cgrp(gr, carry):
            lv = lbuf[pl.ds(w * (N // 16) + gr * 16, 16)]
            for j in range(16):
                cnt[pl.ds(lv[j] * 16, 16)] += ones16
            return carry
        lax.fori_loop(0, N // 16 // 16, cgrp, 0)
        pltpu.sync_copy(cnt, ocnt_hbm.at[w])


def _segment_stats(xg1, labels):
    mesh = plsc.VectorSubcoreMesh(core_axis_name="c", subcore_axis_name="s")
    fn = pl.kernel(
        _segsum_body,
        mesh=mesh,
        out_type=(
            jax.ShapeDtypeStruct((_NG, K * _GW), jnp.float32),
            jax.ShapeDtypeStruct((_NG, K * 16), jnp.float32),
        ),
        scratch_types=[
            pltpu.VMEM((K * _GW,), jnp.float32),
            pltpu.VMEM((K * 16,), jnp.float32),
            pltpu.VMEM((_CHUNK * _GW,), jnp.float32),
            pltpu.VMEM((N,), jnp.int32),
        ],
    )
    osum, ocnt = fn(xg1, labels)
    psums = osum.reshape(_NG, K, _GW)
    pcnts = ocnt.reshape(_NG, K, 16)
    return psums, pcnts


def _update_body(ema_ref, ps_ref, pc_ref, oe_ref, oerr_ref):
    counts = pc_ref[0, :, 0:1]
    for t in range(1, _NG):
        counts = counts + pc_ref[t, :, 0:1]
    dead = counts == 0.0
    errsq = jnp.zeros((), jnp.float32)
    for g in range(_NG):
        s_g = ps_ref[g]
        ema_g = ema_ref[g]
        centers_g = s_g / (counts + EPS)
        centers_g = jnp.where(dead, ema_g, centers_g)
        ema_new_g = ema_g * EMA_DECAY + (1.0 - EMA_DECAY) * centers_g
        diff = ema_new_g - centers_g
        errsq = errsq + jnp.sum(diff * diff)
        oe_ref[g] = ema_new_g
    oerr_ref[...] = jnp.broadcast_to(jnp.sqrt(errsq), (1, 1))


def _update(ema16, psums, pcnts):
    return pl.pallas_call(
        _update_body,
        out_shape=(
            jax.ShapeDtypeStruct((_NG, K, _GW), jnp.float32),
            jax.ShapeDtypeStruct((1, 1), jnp.float32),
        ),
    )(ema16, psums, pcnts)


def kernel(x):
    b, c, t = x.shape
    xf = jnp.transpose(x, (0, 2, 1)).reshape(-1, c)
    n = b * t
    perm = jax.random.permutation(jax.random.key(1), n)
    ema0 = xf[perm[:K], :]
    ema0_16 = ema0.reshape(K, _NG, _GW).transpose(1, 0, 2)
    xg1 = xf.reshape(n, _NG, _GW).transpose(1, 0, 2).reshape(-1)

    def cond_fn(carry):
        i, ema16, labels, broke = carry
        return jnp.logical_and(i < MAX_ITERS, jnp.logical_not(broke))

    def body_fn(carry):
        i, ema16, labels, broke = carry
        ema_std = ema16.transpose(1, 0, 2).reshape(K, C)
        labels = _dist_argmin(xf, ema_std)
        psums, pcnts = _segment_stats(xg1, labels)
        ema16_new, err = _update(ema16, psums, pcnts)
        broke = err[0, 0] < TOL
        return (i + 1, ema16_new, labels, broke)

    init = (jnp.asarray(0, jnp.int32), ema0_16,
            jnp.zeros((n,), jnp.int32), jnp.asarray(False))
    _, ema16_f, labels, broke = lax.while_loop(cond_fn, body_fn, init)

    ema_f = ema16_f.transpose(1, 0, 2).reshape(K, C)
    labels_else = _dist_argmin(xf, ema_f)
    return jnp.where(broke, labels, labels_else)


# speedup vs baseline: 1.3596x; 1.1630x over previous
import functools

import jax
import jax.numpy as jnp
from jax import lax
from jax.experimental import pallas as pl
from jax.experimental.pallas import tpu as pltpu
from jax.experimental.pallas import tpu_sc as plsc

K = 1024
EMA_DECAY = 0.99
TOL = 1e-4
EPS = 1e-5
MAX_ITERS = 4

N = 16384
C = 256
_BM = 1024
_NG = 16
_GW = C // _NG
_CHUNK = 1024
_NCHUNK = N // _CHUNK


def _dist_argmin_body(x_ref, c_ref, o_ref):
    x = x_ref[...]
    c = c_ref[...]
    x2 = jnp.sum(x * x, axis=1, keepdims=True)
    c2 = jnp.sum(c * c, axis=1, keepdims=True).T
    xc = lax.dot_general(x, c, (((1,), (1,)), ((), ())),
                         preferred_element_type=jnp.float32)
    d2 = x2 + c2 - 2.0 * xc
    d = jnp.sqrt(jnp.maximum(d2, 0.0))
    mn = jnp.min(d, axis=1, keepdims=True)
    iota = lax.broadcasted_iota(jnp.int32, d.shape, 1)
    lab = jnp.min(jnp.where(d == mn, iota, jnp.int32(K)), axis=1)
    o_ref[0, 0, :] = lab


def _dist_argmin(xf, centers):
    nb = N // _BM
    out = pl.pallas_call(
        _dist_argmin_body,
        grid=(nb,),
        in_specs=[
            pl.BlockSpec((_BM, C), lambda i: (i, 0)),
            pl.BlockSpec((K, C), lambda i: (0, 0)),
        ],
        out_specs=pl.BlockSpec((1, 1, _BM), lambda i: (i, 0, 0)),
        out_shape=jax.ShapeDtypeStruct((nb, 1, _BM), jnp.int32),
    )(xf, centers)
    return out.reshape(-1)


def _segsum_body(xg_hbm, lab_hbm, osum_hbm, ocnt_hbm, acc, cnt, xbuf, lbuf):
    cid = lax.axis_index("c")
    sid = lax.axis_index("s")
    zeros16 = jnp.zeros((16,), jnp.float32)
    ones16 = jnp.ones((16,), jnp.float32)

    pltpu.sync_copy(lab_hbm, lbuf)

    @pl.when(sid < 8)
    def _sums():
        g = cid * 8 + sid

        def zero_acc(i, carry):
            acc[pl.ds(i * 16, 16)] = zeros16
            return carry
        lax.fori_loop(0, K, zero_acc, 0)

        gbase = g * (N * _GW)

        def chunk_body(ch, carry):
            pltpu.sync_copy(
                xg_hbm.at[pl.ds(gbase + ch * (_CHUNK * _GW), _CHUNK * _GW)],
                xbuf)

            def grp(gr, c2):
                lv = lbuf[pl.ds(ch * _CHUNK + gr * 16, 16)]
                for c in range(_GW):
                    xcol = xbuf[pl.ds(gr * 256 + c * 16, 16)]
                    plsc.addupdate_scatter(acc, [lv + c * K], xcol)
                return c2
            lax.fori_loop(0, _CHUNK // 16, grp, 0)
            return carry
        lax.fori_loop(0, _NCHUNK, chunk_body, 0)
        pltpu.sync_copy(acc, osum_hbm.at[g])

    @pl.when(sid >= 8)
    def _counts():
        w = cid * 8 + (sid - 8)

        def zero_cnt(i, carry):
            cnt[pl.ds(i * 16, 16)] = zeros16
            return carry
        lax.fori_loop(0, K // 16, zero_cnt, 0)

        def cgrp(gr, carry):
            lv = lbuf[pl.ds(w * (N // 16) + gr * 16, 16)]
            plsc.addupdate_scatter(cnt, [lv], ones16)
            return carry
        lax.fori_loop(0, N // 16 // 16, cgrp, 0)
        pltpu.sync_copy(cnt, ocnt_hbm.at[w])


def _segment_stats(xgt, labels):
    mesh = plsc.VectorSubcoreMesh(core_axis_name="c", subcore_axis_name="s")
    fn = pl.kernel(
        _segsum_body,
        mesh=mesh,
        out_type=(
            jax.ShapeDtypeStruct((_NG, _GW * K), jnp.float32),
            jax.ShapeDtypeStruct((_NG, K), jnp.float32),
        ),
        scratch_types=[
            pltpu.VMEM((_GW * K,), jnp.float32),
            pltpu.VMEM((K,), jnp.float32),
            pltpu.VMEM((_CHUNK * _GW,), jnp.float32),
            pltpu.VMEM((N,), jnp.int32),
        ],
        compiler_params=pltpu.CompilerParams(needs_layout_passes=False),
    )
    osum, ocnt = fn(xgt, labels)
    sums_cm = osum.reshape(C, K)
    return sums_cm, ocnt


def _update_body(ema_ref, ps_ref, pc_ref, oe_ref, oerr_ref):
    counts = pc_ref[0:1, :]
    for t in range(1, _NG):
        counts = counts + pc_ref[t:t + 1, :]
    dead = counts == 0.0
    ema = ema_ref[...]
    sums = ps_ref[...]
    centers = sums / (counts + EPS)
    centers = jnp.where(dead, ema, centers)
    ema_new = ema * EMA_DECAY + (1.0 - EMA_DECAY) * centers
    diff = ema_new - centers
    oe_ref[...] = ema_new
    oerr_ref[...] = jnp.broadcast_to(jnp.sqrt(jnp.sum(diff * diff)), (1, 1))


def _update(ema_cm, sums_cm, pcnts):
    return pl.pallas_call(
        _update_body,
        out_shape=(
            jax.ShapeDtypeStruct((C, K), jnp.float32),
            jax.ShapeDtypeStruct((1, 1), jnp.float32),
        ),
    )(ema_cm, sums_cm, pcnts)


def kernel(x):
    b, c, t = x.shape
    xf = jnp.transpose(x, (0, 2, 1)).reshape(-1, c)
    n = b * t
    perm = jax.random.permutation(jax.random.key(1), n)
    ema0 = xf[perm[:K], :]
    ema0_cm = ema0.T
    xgt = (xf.reshape(n // 16, 16, _NG, _GW)
           .transpose(2, 0, 3, 1).reshape(-1))

    def cond_fn(carry):
        i, ema_cm, labels, broke = carry
        return jnp.logical_and(i < MAX_ITERS, jnp.logical_not(broke))

    def body_fn(carry):
        i, ema_cm, labels, broke = carry
        labels = _dist_argmin(xf, ema_cm.T)
        sums_cm, pcnts = _segment_stats(xgt, labels)
        ema_cm_new, err = _update(ema_cm, sums_cm, pcnts)
        broke = err[0, 0] < TOL
        return (i + 1, ema_cm_new, labels, broke)

    init = (jnp.asarray(0, jnp.int32), ema0_cm,
            jnp.zeros((n,), jnp.int32), jnp.asarray(False))
    _, ema_cm_f, labels, broke = lax.while_loop(cond_fn, body_fn, init)

    ema_f = ema_cm_f.T
    labels_else = _dist_argmin(xf, ema_f)
    return jnp.where(broke, labels, labels_else)


# speedup vs baseline: 1.4008x; 1.0303x over previous
import functools

import jax
import jax.numpy as jnp
from jax import lax
from jax.experimental import pallas as pl
from jax.experimental.pallas import tpu as pltpu
from jax.experimental.pallas import tpu_sc as plsc

K = 1024
EMA_DECAY = 0.99
TOL = 1e-4
EPS = 1e-5
MAX_ITERS = 4

N = 16384
C = 256
_BM = 1024
_NG = 16
_GW = C // _NG
_CHUNK = 1024
_NCHUNK = N // _CHUNK


def _dist_argmin_body(x_ref, c_ref, o_ref):
    x = x_ref[...]
    c = c_ref[...]
    x2 = jnp.sum(x * x, axis=1, keepdims=True)
    c2 = jnp.sum(c * c, axis=1, keepdims=True).T
    xc = lax.dot_general(x, c, (((1,), (1,)), ((), ())),
                         preferred_element_type=jnp.float32)
    d2 = x2 + c2 - 2.0 * xc
    d = jnp.sqrt(jnp.maximum(d2, 0.0))
    mn = jnp.min(d, axis=1, keepdims=True)
    iota = lax.broadcasted_iota(jnp.int32, d.shape, 1)
    lab = jnp.min(jnp.where(d == mn, iota, jnp.int32(K)), axis=1)
    o_ref[0, 0, :] = lab


def _dist_argmin(xf, centers):
    nb = N // _BM
    out = pl.pallas_call(
        _dist_argmin_body,
        grid=(nb,),
        in_specs=[
            pl.BlockSpec((_BM, C), lambda i: (i, 0)),
            pl.BlockSpec((K, C), lambda i: (0, 0)),
        ],
        out_specs=pl.BlockSpec((1, 1, _BM), lambda i: (i, 0, 0)),
        out_shape=jax.ShapeDtypeStruct((nb, 1, _BM), jnp.int32),
    )(xf, centers)
    return out.reshape(-1)


def _segsum_body(xg_hbm, lab_hbm, osum_hbm, ocnt_hbm, cnt, xbuf, lbuf, *accs):
    cid = lax.axis_index("c")
    sid = lax.axis_index("s")
    zeros16 = jnp.zeros((16,), jnp.float32)
    ones16 = jnp.ones((16,), jnp.float32)

    pltpu.sync_copy(lab_hbm, lbuf)

    @pl.when(sid < 8)
    def _sums():
        g = cid * 8 + sid

        def zero_acc(i, carry):
            for c in range(_GW):
                accs[c][pl.ds(i * 16, 16)] = zeros16
            return carry
        lax.fori_loop(0, K // 16, zero_acc, 0)

        gbase = g * (N * _GW)

        def chunk_body(ch, carry):
            pltpu.sync_copy(
                xg_hbm.at[pl.ds(gbase + ch * (_CHUNK * _GW), _CHUNK * _GW)],
                xbuf)

            def grp(gr, c2):
                lv = lbuf[pl.ds(ch * _CHUNK + gr * 16, 16)]
                for c in range(_GW):
                    xcol = xbuf[pl.ds(gr * 256 + c * 16, 16)]
                    plsc.addupdate_scatter(accs[c], [lv], xcol)
                return c2
            lax.fori_loop(0, _CHUNK // 16, grp, 0)
            return carry
        lax.fori_loop(0, _NCHUNK, chunk_body, 0)
        for c in range(_GW):
            pltpu.sync_copy(accs[c], osum_hbm.at[g, c])

    @pl.when(sid >= 8)
    def _counts():
        w = cid * 8 + (sid - 8)

        def zero_cnt(i, carry):
            cnt[pl.ds(i * 16, 16)] = zeros16
            return carry
        lax.fori_loop(0, K // 16, zero_cnt, 0)

        def cgrp(gr, carry):
            lv = lbuf[pl.ds(w * (N // 16) + gr * 16, 16)]
            plsc.addupdate_scatter(cnt, [lv], ones16)
            return carry
        lax.fori_loop(0, N // 16 // 16, cgrp, 0)
        pltpu.sync_copy(cnt, ocnt_hbm.at[w])


def _segment_stats(xgt, labels):
    mesh = plsc.VectorSubcoreMesh(core_axis_name="c", subcore_axis_name="s")
    fn = pl.kernel(
        _segsum_body,
        mesh=mesh,
        out_type=(
            jax.ShapeDtypeStruct((_NG, _GW, K), jnp.float32),
            jax.ShapeDtypeStruct((_NG, K), jnp.float32),
        ),
        scratch_types=[
            pltpu.VMEM((K,), jnp.float32),
            pltpu.VMEM((_CHUNK * _GW,), jnp.float32),
            pltpu.VMEM((N,), jnp.int32),
        ] + [pltpu.VMEM((K,), jnp.float32) for _ in range(_GW)],
        compiler_params=pltpu.CompilerParams(needs_layout_passes=False),
    )
    osum, ocnt = fn(xgt, labels)
    sums_cm = osum.reshape(C, K)
    return sums_cm, ocnt


def _update_body(ema_ref, ps_ref, pc_ref, oe_ref, oerr_ref):
    counts = pc_ref[0:1, :]
    for t in range(1, _NG):
        counts = counts + pc_ref[t:t + 1, :]
    dead = counts == 0.0
    ema = ema_ref[...]
    sums = ps_ref[...]
    centers = sums / (counts + EPS)
    centers = jnp.where(dead, ema, centers)
    ema_new = ema * EMA_DECAY + (1.0 - EMA_DECAY) * centers
    diff = ema_new - centers
    oe_ref[...] = ema_new
    oerr_ref[...] = jnp.broadcast_to(jnp.sqrt(jnp.sum(diff * diff)), (1, 1))


def _update(ema_cm, sums_cm, pcnts):
    return pl.pallas_call(
        _update_body,
        out_shape=(
            jax.ShapeDtypeStruct((C, K), jnp.float32),
            jax.ShapeDtypeStruct((1, 1), jnp.float32),
        ),
    )(ema_cm, sums_cm, pcnts)


def kernel(x):
    b, c, t = x.shape
    xf = jnp.transpose(x, (0, 2, 1)).reshape(-1, c)
    n = b * t
    perm = jax.random.permutation(jax.random.key(1), n)
    ema0 = xf[perm[:K], :]
    ema0_cm = ema0.T
    xgt = (xf.reshape(n // 16, 16, _NG, _GW)
           .transpose(2, 0, 3, 1).reshape(-1))

    def cond_fn(carry):
        i, ema_cm, labels, broke = carry
        return jnp.logical_and(i < MAX_ITERS, jnp.logical_not(broke))

    def body_fn(carry):
        i, ema_cm, labels, broke = carry
        labels = _dist_argmin(xf, ema_cm.T)
        sums_cm, pcnts = _segment_stats(xgt, labels)
        ema_cm_new, err = _update(ema_cm, sums_cm, pcnts)
        broke = err[0, 0] < TOL
        return (i + 1, ema_cm_new, labels, broke)

    init = (jnp.asarray(0, jnp.int32), ema0_cm,
            jnp.zeros((n,), jnp.int32), jnp.asarray(False))
    _, ema_cm_f, labels, broke = lax.while_loop(cond_fn, body_fn, init)

    ema_f = ema_cm_f.T
    labels_else = _dist_argmin(xf, ema_f)
    return jnp.where(broke, labels, labels_else)


# speedup vs baseline: 1.4252x; 1.0174x over previous
import functools

import jax
import jax.numpy as jnp
from jax import lax
from jax.experimental import pallas as pl
from jax.experimental.pallas import tpu as pltpu
from jax.experimental.pallas import tpu_sc as plsc

K = 1024
EMA_DECAY = 0.99
TOL = 1e-4
EPS = 1e-5
MAX_ITERS = 4

N = 16384
C = 256
_BM = 1024
_NG = 16
_GW = C // _NG
_CHUNK = 1024
_NCHUNK = N // _CHUNK


def _dist_argmin_body(x_ref, c_ref, o_ref):
    x = x_ref[...]
    c = c_ref[...]
    x2 = jnp.sum(x * x, axis=1, keepdims=True)
    c2 = jnp.sum(c * c, axis=1, keepdims=True).T
    xc = lax.dot_general(x, c, (((1,), (1,)), ((), ())),
                         preferred_element_type=jnp.float32)
    d2 = x2 + c2 - 2.0 * xc
    d = jnp.sqrt(jnp.maximum(d2, 0.0))
    mn = jnp.min(d, axis=1, keepdims=True)
    iota = lax.broadcasted_iota(jnp.int32, d.shape, 1)
    lab = jnp.min(jnp.where(d == mn, iota, jnp.int32(K)), axis=1)
    o_ref[0, 0, :] = lab


def _dist_argmin(xf, centers):
    nb = N // _BM
    out = pl.pallas_call(
        _dist_argmin_body,
        grid=(nb,),
        in_specs=[
            pl.BlockSpec((_BM, C), lambda i: (i, 0)),
            pl.BlockSpec((K, C), lambda i: (0, 0)),
        ],
        out_specs=pl.BlockSpec((1, 1, _BM), lambda i: (i, 0, 0)),
        out_shape=jax.ShapeDtypeStruct((nb, 1, _BM), jnp.int32),
    )(xf, centers)
    return out.reshape(-1)


def _segsum_body(xg_hbm, lab_hbm, osum_hbm, ocnt_hbm, cnt, xbuf, lbuf, *accs):
    cid = lax.axis_index("c")
    sid = lax.axis_index("s")
    zeros16 = jnp.zeros((16,), jnp.float32)
    ones16 = jnp.ones((16,), jnp.float32)

    wid = cid * 16 + sid
    g = lax.rem(wid, _NG)
    half = wid // _NG
    lab_lo = half * (K // 2)
    lab_hi = lab_lo + (K // 2)

    pltpu.sync_copy(lab_hbm, lbuf)

    def zero_acc(i, carry):
        for c in range(_GW):
            accs[c][pl.ds(i * 16, 16)] = zeros16
        cnt[pl.ds(i * 16, 16)] = zeros16
        return carry
    lax.fori_loop(0, K // 16, zero_acc, 0)

    gbase = g * (N * _GW)

    def chunk_body(ch, carry):
        pltpu.sync_copy(
            xg_hbm.at[pl.ds(gbase + ch * (_CHUNK * _GW), _CHUNK * _GW)],
            xbuf)

        def grp(gr, c2):
            lv = lbuf[pl.ds(ch * _CHUNK + gr * 16, 16)]
            m = jnp.logical_and(lv >= lab_lo, lv < lab_hi)
            for c in range(_GW):
                xcol = xbuf[pl.ds(gr * 256 + c * 16, 16)]
                plsc.addupdate_scatter(accs[c], [lv], xcol, mask=m)
            return c2
        lax.fori_loop(0, _CHUNK // 16, grp, 0)
        return carry
    lax.fori_loop(0, _NCHUNK, chunk_body, 0)
    for c in range(_GW):
        pltpu.sync_copy(accs[c].at[pl.ds(lab_lo, K // 2)],
                        osum_hbm.at[g, c, pl.ds(lab_lo, K // 2)])

    def cgrp(gr, carry):
        lv = lbuf[pl.ds(wid * (N // 32) + gr * 16, 16)]
        plsc.addupdate_scatter(cnt, [lv], ones16)
        return carry
    lax.fori_loop(0, N // 32 // 16, cgrp, 0)
    pltpu.sync_copy(cnt, ocnt_hbm.at[wid])


def _segment_stats(xgt, labels):
    mesh = plsc.VectorSubcoreMesh(core_axis_name="c", subcore_axis_name="s")
    fn = pl.kernel(
        _segsum_body,
        mesh=mesh,
        out_type=(
            jax.ShapeDtypeStruct((_NG, _GW, K), jnp.float32),
            jax.ShapeDtypeStruct((32, K), jnp.float32),
        ),
        scratch_types=[
            pltpu.VMEM((K,), jnp.float32),
            pltpu.VMEM((_CHUNK * _GW,), jnp.float32),
            pltpu.VMEM((N,), jnp.int32),
        ] + [pltpu.VMEM((K,), jnp.float32) for _ in range(_GW)],
        compiler_params=pltpu.CompilerParams(needs_layout_passes=False),
    )
    osum, ocnt = fn(xgt, labels)
    sums_cm = osum.reshape(C, K)
    return sums_cm, ocnt


def _update_body(ema_ref, ps_ref, pc_ref, oe_ref, oerr_ref):
    counts = pc_ref[0:1, :]
    for t in range(1, 32):
        counts = counts + pc_ref[t:t + 1, :]
    dead = counts == 0.0
    ema = ema_ref[...]
    sums = ps_ref[...]
    centers = sums / (counts + EPS)
    centers = jnp.where(dead, ema, centers)
    ema_new = ema * EMA_DECAY + (1.0 - EMA_DECAY) * centers
    diff = ema_new - centers
    oe_ref[...] = ema_new
    oerr_ref[...] = jnp.broadcast_to(jnp.sqrt(jnp.sum(diff * diff)), (1, 1))


def _update(ema_cm, sums_cm, pcnts):
    return pl.pallas_call(
        _update_body,
        out_shape=(
            jax.ShapeDtypeStruct((C, K), jnp.float32),
            jax.ShapeDtypeStruct((1, 1), jnp.float32),
        ),
    )(ema_cm, sums_cm, pcnts)


def kernel(x):
    b, c, t = x.shape
    xf = jnp.transpose(x, (0, 2, 1)).reshape(-1, c)
    n = b * t
    perm = jax.random.permutation(jax.random.key(1), n)
    ema0 = xf[perm[:K], :]
    ema0_cm = ema0.T
    xgt = (xf.reshape(n // 16, 16, _NG, _GW)
           .transpose(2, 0, 3, 1).reshape(-1))

    def cond_fn(carry):
        i, ema_cm, labels, broke = carry
        return jnp.logical_and(i < MAX_ITERS, jnp.logical_not(broke))

    def body_fn(carry):
        i, ema_cm, labels, broke = carry
        labels = _dist_argmin(xf, ema_cm.T)
        sums_cm, pcnts = _segment_stats(xgt, labels)
        ema_cm_new, err = _update(ema_cm, sums_cm, pcnts)
        broke = err[0, 0] < TOL
        return (i + 1, ema_cm_new, labels, broke)

    init = (jnp.asarray(0, jnp.int32), ema0_cm,
            jnp.zeros((n,), jnp.int32), jnp.asarray(False))
    _, ema_cm_f, labels, broke = lax.while_loop(cond_fn, body_fn, init)

    ema_f = ema_cm_f.T
    labels_else = _dist_argmin(xf, ema_f)
    return jnp.where(broke, labels, labels_else)


# speedup vs baseline: 1.5481x; 1.0862x over previous
import functools

import jax
import jax.numpy as jnp
from jax import lax
from jax.experimental import pallas as pl
from jax.experimental.pallas import tpu as pltpu
from jax.experimental.pallas import tpu_sc as plsc

K = 1024
EMA_DECAY = 0.99
TOL = 1e-4
EPS = 1e-5
MAX_ITERS = 4

N = 16384
C = 256
_BM = 1024
_NG = 16
_GW = C // _NG
_CHUNK = 1024
_NCHUNK = N // _CHUNK


def _dist_argmin_body(x_ref, c_ref, o_ref):
    x = x_ref[...]
    c = c_ref[...]
    x2 = jnp.sum(x * x, axis=1, keepdims=True)
    c2 = jnp.sum(c * c, axis=1, keepdims=True).T
    xc = lax.dot_general(x, c, (((1,), (1,)), ((), ())),
                         preferred_element_type=jnp.float32)
    d2 = x2 + c2 - 2.0 * xc
    d = jnp.sqrt(jnp.maximum(d2, 0.0))
    mn = jnp.min(d, axis=1, keepdims=True)
    iota = lax.broadcasted_iota(jnp.int32, d.shape, 1)
    lab = jnp.min(jnp.where(d == mn, iota, jnp.int32(K)), axis=1)
    o_ref[0, 0, :] = lab


def _dist_argmin(xf, centers):
    nb = N // _BM
    out = pl.pallas_call(
        _dist_argmin_body,
        grid=(nb,),
        in_specs=[
            pl.BlockSpec((_BM, C), lambda i: (i, 0)),
            pl.BlockSpec((K, C), lambda i: (0, 0)),
        ],
        out_specs=pl.BlockSpec((1, 1, _BM), lambda i: (i, 0, 0)),
        out_shape=jax.ShapeDtypeStruct((nb, 1, _BM), jnp.int32),
    )(xf, centers)
    return out.reshape(-1)


def _segsum_body(xg_hbm, lab_hbm, osum_hbm, ocnt_hbm, cnt, xbuf, xbuf2,
                 lbuf, sem, sem2, *accs):
    cid = lax.axis_index("c")
    sid = lax.axis_index("s")
    zeros16 = jnp.zeros((16,), jnp.float32)
    ones16 = jnp.ones((16,), jnp.float32)

    wid = cid * 16 + sid
    g = lax.rem(wid, _NG)
    half = wid // _NG
    lab_lo = half * (K // 2)
    lab_hi = lab_lo + (K // 2)

    pltpu.sync_copy(lab_hbm, lbuf)

    def zero_acc(i, carry):
        for c in range(_GW):
            accs[c][pl.ds(i * 16, 16)] = zeros16
        cnt[pl.ds(i * 16, 16)] = zeros16
        return carry
    lax.fori_loop(0, K // 16, zero_acc, 0)

    gbase = g * (N * _GW)
    bufs = (xbuf, xbuf2)
    sems = (sem, sem2)

    def _start(ch, b):
        return pltpu.async_copy(
            xg_hbm.at[pl.ds(gbase + ch * (_CHUNK * _GW), _CHUNK * _GW)],
            bufs[b], sems[b])

    def _consume(ch, b):
        buf = bufs[b]

        def grp(gr, c2):
            lv = lbuf[pl.ds(ch * _CHUNK + gr * 16, 16)]
            m = jnp.logical_and(lv >= lab_lo, lv < lab_hi)
            for c in range(_GW):
                xcol = buf[pl.ds(gr * 256 + c * 16, 16)]
                plsc.addupdate_scatter(accs[c], [lv], xcol, mask=m)
            return c2
        lax.fori_loop(0, _CHUNK // 16, grp, 0)

    cp = _start(0, 0)
    for ch in range(_NCHUNK):
        b = ch % 2
        cp.wait()
        if ch + 1 < _NCHUNK:
            cp = _start(ch + 1, 1 - b)
        _consume(ch, b)
    for c in range(_GW):
        pltpu.sync_copy(accs[c].at[pl.ds(lab_lo, K // 2)],
                        osum_hbm.at[g, c, pl.ds(lab_lo, K // 2)])

    def cgrp(gr, carry):
        lv = lbuf[pl.ds(wid * (N // 32) + gr * 16, 16)]
        plsc.addupdate_scatter(cnt, [lv], ones16)
        return carry
    lax.fori_loop(0, N // 32 // 16, cgrp, 0)
    pltpu.sync_copy(cnt, ocnt_hbm.at[wid])


def _segment_stats(xgt, labels):
    mesh = plsc.VectorSubcoreMesh(core_axis_name="c", subcore_axis_name="s")
    fn = pl.kernel(
        _segsum_body,
        mesh=mesh,
        out_type=(
            jax.ShapeDtypeStruct((_NG, _GW, K), jnp.float32),
            jax.ShapeDtypeStruct((32, K), jnp.float32),
        ),
        scratch_types=[
            pltpu.VMEM((K,), jnp.float32),
            pltpu.VMEM((_CHUNK * _GW,), jnp.float32),
            pltpu.VMEM((_CHUNK * _GW,), jnp.float32),
            pltpu.VMEM((N,), jnp.int32),
            pltpu.SemaphoreType.DMA,
            pltpu.SemaphoreType.DMA,
        ] + [pltpu.VMEM((K,), jnp.float32) for _ in range(_GW)],
        compiler_params=pltpu.CompilerParams(needs_layout_passes=False),
    )
    osum, ocnt = fn(xgt, labels)
    sums_cm = osum.reshape(C, K)
    return sums_cm, ocnt


def _update_body(ema_ref, ps_ref, pc_ref, oe_ref, oet_ref, oerr_ref):
    counts = pc_ref[0:1, :]
    for t in range(1, 32):
        counts = counts + pc_ref[t:t + 1, :]
    dead = counts == 0.0
    ema = ema_ref[...]
    sums = ps_ref[...]
    centers = sums / (counts + EPS)
    centers = jnp.where(dead, ema, centers)
    ema_new = ema * EMA_DECAY + (1.0 - EMA_DECAY) * centers
    diff = ema_new - centers
    oe_ref[...] = ema_new
    oet_ref[...] = ema_new.T
    oerr_ref[...] = jnp.broadcast_to(jnp.sqrt(jnp.sum(diff * diff)), (1, 1))


def _update(ema_cm, sums_cm, pcnts):
    return pl.pallas_call(
        _update_body,
        out_shape=(
            jax.ShapeDtypeStruct((C, K), jnp.float32),
            jax.ShapeDtypeStruct((K, C), jnp.float32),
            jax.ShapeDtypeStruct((1, 1), jnp.float32),
        ),
    )(ema_cm, sums_cm, pcnts)


def kernel(x):
    b, c, t = x.shape
    xf = jnp.transpose(x, (0, 2, 1)).reshape(-1, c)
    n = b * t
    perm = jax.random.permutation(jax.random.key(1), n)
    ema0 = xf[perm[:K], :]
    ema0_cm = ema0.T
    xgt = (xf.reshape(n // 16, 16, _NG, _GW)
           .transpose(2, 0, 3, 1).reshape(-1))

    def cond_fn(carry):
        i, ema_cm, ema_rm, labels, broke = carry
        return jnp.logical_and(i < MAX_ITERS, jnp.logical_not(broke))

    def body_fn(carry):
        i, ema_cm, ema_rm, labels, broke = carry
        labels = _dist_argmin(xf, ema_rm)
        sums_cm, pcnts = _segment_stats(xgt, labels)
        ema_cm_new, ema_rm_new, err = _update(ema_cm, sums_cm, pcnts)
        broke = err[0, 0] < TOL
        return (i + 1, ema_cm_new, ema_rm_new, labels, broke)

    init = (jnp.asarray(0, jnp.int32), ema0_cm, ema0,
            jnp.zeros((n,), jnp.int32), jnp.asarray(False))
    _, _, ema_f, labels, broke = lax.while_loop(cond_fn, body_fn, init)
    labels_else = _dist_argmin(xf, ema_f)
    return jnp.where(broke, labels, labels_else)


# speedup vs baseline: 1.9480x; 1.2583x over previous
import functools

import jax
import jax.numpy as jnp
from jax import lax
from jax.experimental import pallas as pl
from jax.experimental.pallas import tpu as pltpu
from jax.experimental.pallas import tpu_sc as plsc

K = 1024
EMA_DECAY = 0.99
TOL = 1e-4
EPS = 1e-5
MAX_ITERS = 4

N = 16384
C = 256
_BM = 1024
_NG = 16
_GW = C // _NG
_CHUNK = 1024
_NCHUNK = N // _CHUNK


def _dist_argmin_body(x_ref, c_ref, o_ref):
    x = x_ref[...]
    c = c_ref[...]
    x2 = jnp.sum(x * x, axis=1, keepdims=True)
    c2 = jnp.sum(c * c, axis=1, keepdims=True).T
    xc = lax.dot_general(x, c, (((1,), (1,)), ((), ())),
                         preferred_element_type=jnp.float32)
    d2 = x2 + c2 - 2.0 * xc
    d = jnp.sqrt(jnp.maximum(d2, 0.0))
    mn = jnp.min(d, axis=1, keepdims=True)
    iota = lax.broadcasted_iota(jnp.int32, d.shape, 1)
    lab = jnp.min(jnp.where(d == mn, iota, jnp.int32(K)), axis=1)
    o_ref[0, 0, :] = lab


def _dist_argmin(xf, centers):
    nb = N // _BM
    out = pl.pallas_call(
        _dist_argmin_body,
        grid=(nb,),
        in_specs=[
            pl.BlockSpec((_BM, C), lambda i: (i, 0)),
            pl.BlockSpec((K, C), lambda i: (0, 0)),
        ],
        out_specs=pl.BlockSpec((1, 1, _BM), lambda i: (i, 0, 0)),
        out_shape=jax.ShapeDtypeStruct((nb, 1, _BM), jnp.int32),
    )(xf, centers)
    return out.reshape(-1)


def _segsum_body(xg_hbm, lab_hbm, osum_hbm, ocnt_hbm, cnt, xbuf, xbuf2,
                 lbuf, sem, sem2, *accs):
    cid = lax.axis_index("c")
    sid = lax.axis_index("s")
    zeros16 = jnp.zeros((16,), jnp.float32)
    ones16 = jnp.ones((16,), jnp.float32)

    wid = cid * 16 + sid
    g = lax.rem(wid, _NG)
    half = wid // _NG
    lab_lo = half * (K // 2)
    lab_hi = lab_lo + (K // 2)

    pltpu.sync_copy(lab_hbm, lbuf)

    def zero_acc(i, carry):
        for c in range(_GW):
            accs[c][pl.ds(i * 16, 16)] = zeros16
        cnt[pl.ds(i * 16, 16)] = zeros16
        return carry
    lax.fori_loop(0, K // 16, zero_acc, 0)

    bufs = (xbuf, xbuf2)
    sems = (sem, sem2)

    def _start(ch, b):
        return pltpu.async_copy(
            xg_hbm.at[pl.ds(g * _GW, _GW), pl.ds(ch * _CHUNK, _CHUNK)],
            bufs[b], sems[b])

    def _consume(ch, b):
        buf = bufs[b]

        def grp(gr, c2):
            lv = lbuf[pl.ds(ch * _CHUNK + gr * 16, 16)]
            m = jnp.logical_and(lv >= lab_lo, lv < lab_hi)
            for c in range(_GW):
                xcol = buf[c, pl.ds(gr * 16, 16)]
                plsc.addupdate_scatter(accs[c], [lv], xcol, mask=m)
            return c2
        lax.fori_loop(0, _CHUNK // 16, grp, 0)

    cp = _start(0, 0)
    for ch in range(_NCHUNK):
        b = ch % 2
        cp.wait()
        if ch + 1 < _NCHUNK:
            cp = _start(ch + 1, 1 - b)
        _consume(ch, b)
    for c in range(_GW):
        pltpu.sync_copy(accs[c].at[pl.ds(lab_lo, K // 2)],
                        osum_hbm.at[g, c, pl.ds(lab_lo, K // 2)])

    def cgrp(gr, carry):
        lv = lbuf[pl.ds(wid * (N // 32) + gr * 16, 16)]
        plsc.addupdate_scatter(cnt, [lv], ones16)
        return carry
    lax.fori_loop(0, N // 32 // 16, cgrp, 0)
    pltpu.sync_copy(cnt, ocnt_hbm.at[wid])


def _segment_stats(xgt, labels):
    mesh = plsc.VectorSubcoreMesh(core_axis_name="c", subcore_axis_name="s")
    fn = pl.kernel(
        _segsum_body,
        mesh=mesh,
        out_type=(
            jax.ShapeDtypeStruct((_NG, _GW, K), jnp.float32),
            jax.ShapeDtypeStruct((32, K), jnp.float32),
        ),
        scratch_types=[
            pltpu.VMEM((K,), jnp.float32),
            pltpu.VMEM((_GW, _CHUNK), jnp.float32),
            pltpu.VMEM((_GW, _CHUNK), jnp.float32),
            pltpu.VMEM((N,), jnp.int32),
            pltpu.SemaphoreType.DMA,
            pltpu.SemaphoreType.DMA,
        ] + [pltpu.VMEM((K,), jnp.float32) for _ in range(_GW)],
        compiler_params=pltpu.CompilerParams(needs_layout_passes=False),
    )
    osum, ocnt = fn(xgt, labels)
    sums_cm = osum.reshape(C, K)
    return sums_cm, ocnt


def _update_body(ema_ref, ps_ref, pc_ref, oe_ref, oet_ref, oerr_ref):
    counts = pc_ref[0:1, :]
    for t in range(1, 32):
        counts = counts + pc_ref[t:t + 1, :]
    dead = counts == 0.0
    ema = ema_ref[...]
    sums = ps_ref[...]
    centers = sums / (counts + EPS)
    centers = jnp.where(dead, ema, centers)
    ema_new = ema * EMA_DECAY + (1.0 - EMA_DECAY) * centers
    diff = ema_new - centers
    oe_ref[...] = ema_new
    oet_ref[...] = ema_new.T
    oerr_ref[...] = jnp.broadcast_to(jnp.sqrt(jnp.sum(diff * diff)), (1, 1))


def _update(ema_cm, sums_cm, pcnts):
    return pl.pallas_call(
        _update_body,
        out_shape=(
            jax.ShapeDtypeStruct((C, K), jnp.float32),
            jax.ShapeDtypeStruct((K, C), jnp.float32),
            jax.ShapeDtypeStruct((1, 1), jnp.float32),
        ),
    )(ema_cm, sums_cm, pcnts)


def kernel(x):
    b, c, t = x.shape
    xf = jnp.transpose(x, (0, 2, 1)).reshape(-1, c)
    n = b * t
    perm = jax.random.permutation(jax.random.key(1), n)
    ema0 = xf[perm[:K], :]
    ema0_cm = ema0.T
    xgt = jnp.transpose(x, (1, 0, 2)).reshape(c, n)

    def cond_fn(carry):
        i, ema_cm, ema_rm, labels, broke = carry
        return jnp.logical_and(i < MAX_ITERS, jnp.logical_not(broke))

    def body_fn(carry):
        i, ema_cm, ema_rm, labels, broke = carry
        labels = _dist_argmin(xf, ema_rm)
        sums_cm, pcnts = _segment_stats(xgt, labels)
        ema_cm_new, ema_rm_new, err = _update(ema_cm, sums_cm, pcnts)
        broke = err[0, 0] < TOL
        return (i + 1, ema_cm_new, ema_rm_new, labels, broke)

    init = (jnp.asarray(0, jnp.int32), ema0_cm, ema0,
            jnp.zeros((n,), jnp.int32), jnp.asarray(False))
    _, _, ema_f, labels, broke = lax.while_loop(cond_fn, body_fn, init)
    labels_else = _dist_argmin(xf, ema_f)
    return jnp.where(broke, labels, labels_else)


# speedup vs baseline: 1.9527x; 1.0024x over previous
import functools

import jax
import jax.numpy as jnp
from jax import lax
from jax.experimental import pallas as pl
from jax.experimental.pallas import tpu as pltpu
from jax.experimental.pallas import tpu_sc as plsc

K = 1024
EMA_DECAY = 0.99
TOL = 1e-4
EPS = 1e-5
MAX_ITERS = 4

N = 16384
C = 256
_BM = 1024
_NG = 16
_GW = C // _NG
_CHUNK = 1024
_NCHUNK = N // _CHUNK


def _dist_argmin_body(x_ref, c_ref, o_ref):
    x = x_ref[...]
    c = c_ref[...]
    x2 = jnp.sum(x * x, axis=1, keepdims=True)
    c2 = jnp.sum(c * c, axis=1, keepdims=True).T
    xc = lax.dot_general(x, c, (((1,), (1,)), ((), ())),
                         preferred_element_type=jnp.float32)
    d2 = x2 + c2 - 2.0 * xc
    d = jnp.sqrt(jnp.maximum(d2, 0.0))
    mn = jnp.min(d, axis=1, keepdims=True)
    iota = lax.broadcasted_iota(jnp.int32, d.shape, 1)
    lab = jnp.min(jnp.where(d == mn, iota, jnp.int32(K)), axis=1)
    o_ref[0, 0, :] = lab


def _dist_argmin(xf, centers):
    nb = N // _BM
    out = pl.pallas_call(
        _dist_argmin_body,
        grid=(nb,),
        in_specs=[
            pl.BlockSpec((_BM, C), lambda i: (i, 0)),
            pl.BlockSpec((K, C), lambda i: (0, 0)),
        ],
        out_specs=pl.BlockSpec((1, 1, _BM), lambda i: (i, 0, 0)),
        out_shape=jax.ShapeDtypeStruct((nb, 1, _BM), jnp.int32),
    )(xf, centers)
    return out.reshape(-1)


def _segsum_body(xg_hbm, lab_hbm, osum_hbm, ocnt_hbm, cnt, xbuf, xbuf2,
                 lbuf, sem, sem2, *accs):
    cid = lax.axis_index("c")
    sid = lax.axis_index("s")
    zeros16 = jnp.zeros((16,), jnp.float32)
    ones16 = jnp.ones((16,), jnp.float32)

    wid = cid * 16 + sid
    g = lax.rem(wid, _NG)
    half = wid // _NG
    lab_lo = half * (K // 2)
    lab_hi = lab_lo + (K // 2)

    pltpu.sync_copy(lab_hbm, lbuf)

    def zero_acc(i, carry):
        for c in range(_GW):
            accs[c][pl.ds(i * 16, 16)] = zeros16
        cnt[pl.ds(i * 16, 16)] = zeros16
        return carry
    lax.fori_loop(0, K // 16, zero_acc, 0)

    bufs = (xbuf, xbuf2)
    sems = (sem, sem2)

    def _start(ch, b):
        return pltpu.async_copy(
            xg_hbm.at[pl.ds(g * _GW, _GW), pl.ds(ch * _CHUNK, _CHUNK)],
            bufs[b], sems[b])

    def _consume(ch, b):
        buf = bufs[b]

        def grp(gr2, c2):
            for u in range(2):
                gr = gr2 * 2 + u
                lv = lbuf[pl.ds(ch * _CHUNK + gr * 16, 16)]
                m = jnp.logical_and(lv >= lab_lo, lv < lab_hi)
                for c in range(_GW):
                    xcol = buf[c, pl.ds(gr * 16, 16)]
                    plsc.addupdate_scatter(accs[c], [lv], xcol, mask=m)
            return c2
        lax.fori_loop(0, _CHUNK // 32, grp, 0)

    cp = _start(0, 0)
    for ch in range(_NCHUNK):
        b = ch % 2
        cp.wait()
        if ch + 1 < _NCHUNK:
            cp = _start(ch + 1, 1 - b)
        _consume(ch, b)
    for c in range(_GW):
        pltpu.sync_copy(accs[c].at[pl.ds(lab_lo, K // 2)],
                        osum_hbm.at[g, c, pl.ds(lab_lo, K // 2)])

    def cgrp(gr, carry):
        lv = lbuf[pl.ds(wid * (N // 32) + gr * 16, 16)]
        plsc.addupdate_scatter(cnt, [lv], ones16)
        return carry
    lax.fori_loop(0, N // 32 // 16, cgrp, 0)
    pltpu.sync_copy(cnt, ocnt_hbm.at[wid])


def _segment_stats(xgt, labels):
    mesh = plsc.VectorSubcoreMesh(core_axis_name="c", subcore_axis_name="s")
    fn = pl.kernel(
        _segsum_body,
        mesh=mesh,
        out_type=(
            jax.ShapeDtypeStruct((_NG, _GW, K), jnp.float32),
            jax.ShapeDtypeStruct((32, K), jnp.float32),
        ),
        scratch_types=[
            pltpu.VMEM((K,), jnp.float32),
            pltpu.VMEM((_GW, _CHUNK), jnp.float32),
            pltpu.VMEM((_GW, _CHUNK), jnp.float32),
            pltpu.VMEM((N,), jnp.int32),
            pltpu.SemaphoreType.DMA,
            pltpu.SemaphoreType.DMA,
        ] + [pltpu.VMEM((K,), jnp.float32) for _ in range(_GW)],
        compiler_params=pltpu.CompilerParams(needs_layout_passes=False),
    )
    osum, ocnt = fn(xgt, labels)
    sums_cm = osum.reshape(C, K)
    return sums_cm, ocnt


def _update_body(ema_ref, ps_ref, pc_ref, oe_ref, oet_ref, oerr_ref):
    counts = pc_ref[0:1, :]
    for t in range(1, 32):
        counts = counts + pc_ref[t:t + 1, :]
    dead = counts == 0.0
    ema = ema_ref[...]
    sums = ps_ref[...]
    centers = sums / (counts + EPS)
    centers = jnp.where(dead, ema, centers)
    ema_new = ema * EMA_DECAY + (1.0 - EMA_DECAY) * centers
    diff = ema_new - centers
    oe_ref[...] = ema_new
    oet_ref[...] = ema_new.T
    oerr_ref[...] = jnp.broadcast_to(jnp.sqrt(jnp.sum(diff * diff)), (1, 1))


def _update(ema_cm, sums_cm, pcnts):
    return pl.pallas_call(
        _update_body,
        out_shape=(
            jax.ShapeDtypeStruct((C, K), jnp.float32),
            jax.ShapeDtypeStruct((K, C), jnp.float32),
            jax.ShapeDtypeStruct((1, 1), jnp.float32),
        ),
    )(ema_cm, sums_cm, pcnts)


def kernel(x):
    b, c, t = x.shape
    xf = jnp.transpose(x, (0, 2, 1)).reshape(-1, c)
    n = b * t
    perm = jax.random.permutation(jax.random.key(1), n)
    ema0 = xf[perm[:K], :]
    ema0_cm = ema0.T
    xgt = jnp.transpose(x, (1, 0, 2)).reshape(c, n)

    def cond_fn(carry):
        i, ema_cm, ema_rm, labels, broke = carry
        return jnp.logical_and(i < MAX_ITERS, jnp.logical_not(broke))

    def body_fn(carry):
        i, ema_cm, ema_rm, labels, broke = carry
        labels = _dist_argmin(xf, ema_rm)
        sums_cm, pcnts = _segment_stats(xgt, labels)
        ema_cm_new, ema_rm_new, err = _update(ema_cm, sums_cm, pcnts)
        broke = err[0, 0] < TOL
        return (i + 1, ema_cm_new, ema_rm_new, labels, broke)

    init = (jnp.asarray(0, jnp.int32), ema0_cm, ema0,
            jnp.zeros((n,), jnp.int32), jnp.asarray(False))
    _, _, ema_f, labels, broke = lax.while_loop(cond_fn, body_fn, init)
    labels_else = _dist_argmin(xf, ema_f)
    return jnp.where(broke, labels, labels_else)
